# Initial kernel scaffold; baseline (speedup 1.0000x reference)
import jax
import jax.numpy as jnp
from jax.experimental import pallas as pl


def kernel(output_imgs, char_target, aff_target):
    raise NotImplementedError("write your pallas kernel here")


# speedup vs baseline: 41.4282x; 41.4282x over previous
import functools

import jax
import jax.numpy as jnp
from jax import lax
from jax.experimental import pallas as pl
from jax.experimental.pallas import tpu as pltpu
from jax.experimental.pallas import tpu_sc as plsc

NC, NS, L = 2, 16, 16
NW = NC * NS
H = W = 512
N = H * W
CHUNK = 16384
NCHUNK = N // CHUNK
VSTEPS = CHUNK // L


def _make_reduce_kernel():
    mesh = plsc.VectorSubcoreMesh(
        core_axis_name="c", subcore_axis_name="s",
        num_cores=NC, num_subcores=NS)

    @functools.partial(
        pl.kernel,
        out_type=jax.ShapeDtypeStruct((NW, 4, L), jnp.float32),
        mesh=mesh,
        scratch_types=[
            pltpu.VMEM((2, CHUNK), jnp.float32),
            pltpu.VMEM((2, CHUNK), jnp.float32),
            pltpu.VMEM((4, L), jnp.float32),
        ],
    )
    def reduce_kernel(pred_hbm, char_hbm, aff_hbm, out_hbm, pbuf, tbuf, ostage):
        w = lax.axis_index("s") * NC + lax.axis_index("c")
        samp = w // 2
        chan = w % 2

        def load_chunk(g, slot):
            off = g * CHUNK
            pltpu.sync_copy(pred_hbm.at[w, pl.ds(off, CHUNK)], pbuf.at[slot])

            @pl.when(chan == 0)
            def _():
                pltpu.sync_copy(char_hbm.at[samp, pl.ds(off, CHUNK)],
                                tbuf.at[slot])

            @pl.when(chan == 1)
            def _():
                pltpu.sync_copy(aff_hbm.at[samp, pl.ds(off, CHUNK)],
                                tbuf.at[slot])

        def chunk_body(g, carry):
            load_chunk(g, 0)

            def vstep(i, c2):
                aa, ap, ct = c2
                p = pbuf[0, pl.ds(i * L, L)]
                t = tbuf[0, pl.ds(i * L, L)]
                d = p - t
                l = d * d
                m = t > 0.0
                aa = aa + l
                ap = ap + jnp.where(m, l, 0.0)
                ct = ct + jnp.where(m, 1.0, 0.0)
                return (aa, ap, ct)

            return lax.fori_loop(0, VSTEPS, vstep, carry)

        zeros = jnp.zeros((L,), jnp.float32)
        acc_all, acc_pos, cnt = lax.fori_loop(
            0, NCHUNK, chunk_body, (zeros, zeros, zeros))
        ostage[0, :] = acc_all
        ostage[1, :] = acc_pos
        ostage[2, :] = cnt
        ostage[3, :] = zeros
        pltpu.sync_copy(ostage, out_hbm.at[w])

    return reduce_kernel


_reduce = _make_reduce_kernel()


def _topk_tc_kernel(k_sref, pred_ref, tgt_ref, out_ref):
    w = pl.program_id(0)
    k = k_sref[w]
    pred = pred_ref[0]
    tgt = tgt_ref[0]
    loss = (pred - tgt) ** 2
    keys = jnp.where(tgt > 0.0, jnp.int32(-1),
                     lax.bitcast_convert_type(loss, jnp.int32))

    def bit_step(i, t):
        cand = t | (jnp.int32(1) << (30 - i))
        cnt = jnp.sum((keys >= cand).astype(jnp.int32))
        return jnp.where(cnt >= k, cand, t)

    t = lax.fori_loop(0, 31, bit_step, jnp.int32(0))
    cnt_gt = jnp.sum((keys > t).astype(jnp.int32))
    sum_gt = jnp.sum(jnp.where(keys > t, loss, 0.0))
    tval = lax.bitcast_convert_type(t, jnp.float32)
    res = sum_gt + (k - cnt_gt).astype(jnp.float32) * tval
    out_ref[0] = jnp.full((8, 128), res, jnp.float32)


def _exact_topk_sums(kk, pred32, char_target, aff_target):
    tgt32 = jnp.stack([char_target, aff_target], axis=1).reshape(NW, H, W)
    grid_spec = pltpu.PrefetchScalarGridSpec(
        num_scalar_prefetch=1,
        grid=(NW,),
        in_specs=[
            pl.BlockSpec((1, H, W), lambda w, k: (w, 0, 0)),
            pl.BlockSpec((1, H, W), lambda w, k: (w, 0, 0)),
        ],
        out_specs=pl.BlockSpec((1, 8, 128), lambda w, k: (w, 0, 0)),
    )
    out = pl.pallas_call(
        _topk_tc_kernel,
        grid_spec=grid_spec,
        out_shape=jax.ShapeDtypeStruct((NW, 8, 128), jnp.float32),
    )(kk, pred32.reshape(NW, H, W), tgt32)
    return out[:, 0, 0]


def kernel(output_imgs, char_target, aff_target):
    B = output_imgs.shape[0]
    pred32 = output_imgs.reshape(NW, N)
    charf = char_target.reshape(B, N)
    afff = aff_target.reshape(B, N)

    accs = _reduce(pred32, charf, afff)
    sums = jnp.sum(accs, axis=-1)
    sum_all = sums[:, 0]
    sum_pos = sums[:, 1]
    num_pos_f = sums[:, 2]
    num_pos = num_pos_f.astype(jnp.int32)

    m = jnp.int32(N) - num_pos
    k = jnp.minimum(num_pos * 3, m)
    kf = k.astype(jnp.float32)

    mean_all = sum_all / jnp.float32(N)
    positive_mean = sum_pos / num_pos_f
    easy_topk_mean = (sum_all - sum_pos) / kf

    need_hard = jnp.any((num_pos * 3 < m) & (k >= 10))
    hard_sums = lax.cond(
        need_hard,
        lambda: _exact_topk_sums(k, pred32, char_target, aff_target),
        lambda: jnp.zeros((NW,), jnp.float32),
    )
    topk_mean = jnp.where(num_pos * 3 >= m, easy_topk_mean, hard_sums / kf)
    ohem = positive_mean + topk_mean
    pair_loss = jnp.where(k < 10, mean_all, ohem)
    return jnp.sum(pair_loss) / jnp.float32(B)


# speedup vs baseline: 57.6344x; 1.3912x over previous
import functools

import jax
import jax.numpy as jnp
from jax import lax
from jax.experimental import pallas as pl
from jax.experimental.pallas import tpu as pltpu
from jax.experimental.pallas import tpu_sc as plsc

NC, NS, L = 2, 16, 16
NW = NC * NS
H = W = 512
N = H * W
CHUNK = 16384
NCHUNK = N // CHUNK
VSTEPS = CHUNK // L


def _make_reduce_kernel():
    mesh = plsc.VectorSubcoreMesh(
        core_axis_name="c", subcore_axis_name="s",
        num_cores=NC, num_subcores=NS)

    @functools.partial(
        pl.kernel,
        out_type=jax.ShapeDtypeStruct((NW, 4, L), jnp.float32),
        mesh=mesh,
        scratch_types=[
            pltpu.VMEM((2, CHUNK), jnp.float32),
            pltpu.VMEM((2, CHUNK), jnp.float32),
            pltpu.VMEM((4, L), jnp.float32),
            pltpu.SemaphoreType.DMA,
            pltpu.SemaphoreType.DMA,
            pltpu.SemaphoreType.DMA,
            pltpu.SemaphoreType.DMA,
        ],
    )
    def reduce_kernel(pred_hbm, char_hbm, aff_hbm, out_hbm, pbuf, tbuf,
                      ostage, ps0, ps1, ts0, ts1):
        w = lax.axis_index("s") * NC + lax.axis_index("c")
        samp = w // 2
        chan = w % 2
        psem = (ps0, ps1)
        tsem = (ts0, ts1)

        def start_load(g, slot):
            off = g * CHUNK
            pltpu.make_async_copy(pred_hbm.at[w, pl.ds(off, CHUNK)],
                                  pbuf.at[slot], psem[slot]).start()

            @pl.when(chan == 0)
            def _():
                pltpu.make_async_copy(char_hbm.at[samp, pl.ds(off, CHUNK)],
                                      tbuf.at[slot], tsem[slot]).start()

            @pl.when(chan == 1)
            def _():
                pltpu.make_async_copy(aff_hbm.at[samp, pl.ds(off, CHUNK)],
                                      tbuf.at[slot], tsem[slot]).start()

        def wait_load(slot):
            pltpu.make_async_copy(pred_hbm.at[0, pl.ds(0, CHUNK)],
                                  pbuf.at[slot], psem[slot]).wait()
            pltpu.make_async_copy(char_hbm.at[0, pl.ds(0, CHUNK)],
                                  tbuf.at[slot], tsem[slot]).wait()

        UNROLL = 8

        def compute(slot, carry):
            def vstep(i, c2):
                aa, ap, ct = c2
                base = i * (L * UNROLL)
                ls, ms = [], []
                for u in range(UNROLL):
                    p = pbuf[slot, pl.ds(base + u * L, L)]
                    t = tbuf[slot, pl.ds(base + u * L, L)]
                    d = p - t
                    ls.append(d * d)
                    ms.append(t > 0.0)
                lp = [jnp.where(m, l, 0.0) for m, l in zip(ms, ls)]
                lc = [jnp.where(m, 1.0, 0.0) for m in ms]
                def tree(xs):
                    while len(xs) > 1:
                        xs = [xs[j] + xs[j + 1] for j in range(0, len(xs), 2)]
                    return xs[0]
                return (aa + tree(ls), ap + tree(lp), ct + tree(lc))

            return lax.fori_loop(0, VSTEPS // UNROLL, vstep, carry)

        zeros = jnp.zeros((L,), jnp.float32)
        carry = (zeros, zeros, zeros)
        start_load(0, 0)
        def super_body(gg, carry):
            start_load(2 * gg + 1, 1)
            wait_load(0)
            carry = compute(0, carry)

            @pl.when(gg < NCHUNK // 2 - 1)
            def _():
                start_load(2 * gg + 2, 0)

            wait_load(1)
            return compute(1, carry)

        acc_all, acc_pos, cnt = lax.fori_loop(0, NCHUNK // 2, super_body, carry)
        ostage[0, :] = acc_all
        ostage[1, :] = acc_pos
        ostage[2, :] = cnt
        ostage[3, :] = zeros
        pltpu.sync_copy(ostage, out_hbm.at[w])

    return reduce_kernel


_reduce = _make_reduce_kernel()


def _topk_tc_kernel(k_sref, pred_ref, tgt_ref, out_ref):
    w = pl.program_id(0)
    k = k_sref[w]
    pred = pred_ref[0]
    tgt = tgt_ref[0]
    loss = (pred - tgt) ** 2
    keys = jnp.where(tgt > 0.0, jnp.int32(-1),
                     lax.bitcast_convert_type(loss, jnp.int32))

    def bit_step(i, t):
        cand = t | (jnp.int32(1) << (30 - i))
        cnt = jnp.sum((keys >= cand).astype(jnp.int32))
        return jnp.where(cnt >= k, cand, t)

    t = lax.fori_loop(0, 31, bit_step, jnp.int32(0))
    cnt_gt = jnp.sum((keys > t).astype(jnp.int32))
    sum_gt = jnp.sum(jnp.where(keys > t, loss, 0.0))
    tval = lax.bitcast_convert_type(t, jnp.float32)
    res = sum_gt + (k - cnt_gt).astype(jnp.float32) * tval
    out_ref[0] = jnp.full((8, 128), res, jnp.float32)


def _exact_topk_sums(kk, pred32, char_target, aff_target):
    tgt32 = jnp.stack([char_target, aff_target], axis=1).reshape(NW, H, W)
    grid_spec = pltpu.PrefetchScalarGridSpec(
        num_scalar_prefetch=1,
        grid=(NW,),
        in_specs=[
            pl.BlockSpec((1, H, W), lambda w, k: (w, 0, 0)),
            pl.BlockSpec((1, H, W), lambda w, k: (w, 0, 0)),
        ],
        out_specs=pl.BlockSpec((1, 8, 128), lambda w, k: (w, 0, 0)),
    )
    out = pl.pallas_call(
        _topk_tc_kernel,
        grid_spec=grid_spec,
        out_shape=jax.ShapeDtypeStruct((NW, 8, 128), jnp.float32),
    )(kk, pred32.reshape(NW, H, W), tgt32)
    return out[:, 0, 0]


def kernel(output_imgs, char_target, aff_target):
    B = output_imgs.shape[0]
    pred32 = output_imgs.reshape(NW, N)
    charf = char_target.reshape(B, N)
    afff = aff_target.reshape(B, N)

    accs = _reduce(pred32, charf, afff)
    sums = jnp.sum(accs, axis=-1)
    sum_all = sums[:, 0]
    sum_pos = sums[:, 1]
    num_pos_f = sums[:, 2]
    num_pos = num_pos_f.astype(jnp.int32)

    m = jnp.int32(N) - num_pos
    k = jnp.minimum(num_pos * 3, m)
    kf = k.astype(jnp.float32)

    mean_all = sum_all / jnp.float32(N)
    positive_mean = sum_pos / num_pos_f
    easy_topk_mean = (sum_all - sum_pos) / kf

    need_hard = jnp.any((num_pos * 3 < m) & (k >= 10))
    hard_sums = lax.cond(
        need_hard,
        lambda: _exact_topk_sums(k, pred32, char_target, aff_target),
        lambda: jnp.zeros((NW,), jnp.float32),
    )
    topk_mean = jnp.where(num_pos * 3 >= m, easy_topk_mean, hard_sums / kf)
    ohem = positive_mean + topk_mean
    pair_loss = jnp.where(k < 10, mean_all, ohem)
    return jnp.sum(pair_loss) / jnp.float32(B)


# speedup vs baseline: 76.6921x; 1.3307x over previous
import functools

import jax
import jax.numpy as jnp
from jax import lax
from jax.experimental import pallas as pl
from jax.experimental.pallas import tpu as pltpu
from jax.experimental.pallas import tpu_sc as plsc

NC, NS, L = 2, 16, 16
NW = NC * NS
H = W = 512
N = H * W
ROWS = 32
NCHUNK = H // ROWS


def _make_reduce_kernel():
    mesh = plsc.VectorSubcoreMesh(
        core_axis_name="c", subcore_axis_name="s",
        num_cores=NC, num_subcores=NS)

    @functools.partial(
        pl.kernel,
        out_type=jax.ShapeDtypeStruct((NW, 8, 128), jnp.float32),
        mesh=mesh,
        compiler_params=pltpu.CompilerParams(use_tc_tiling_on_sc=True),
        scratch_types=[
            pltpu.VMEM((2, ROWS, W), jnp.float32),
            pltpu.VMEM((2, ROWS, W), jnp.float32),
            pltpu.VMEM((8, 128), jnp.float32),
            pltpu.SemaphoreType.DMA,
            pltpu.SemaphoreType.DMA,
            pltpu.SemaphoreType.DMA,
            pltpu.SemaphoreType.DMA,
        ],
    )
    def reduce_kernel(pred_hbm, char_hbm, aff_hbm, out_hbm, pbuf, tbuf,
                      ostage, ps0, ps1, ts0, ts1):
        w = lax.axis_index("s") * NC + lax.axis_index("c")
        samp = w // 2
        chan = w % 2
        psem = (ps0, ps1)
        tsem = (ts0, ts1)

        def start_load(g, slot):
            r0 = g * ROWS
            pltpu.make_async_copy(pred_hbm.at[samp, chan, pl.ds(r0, ROWS), :],
                                  pbuf.at[slot], psem[slot]).start()

            @pl.when(chan == 0)
            def _():
                pltpu.make_async_copy(char_hbm.at[samp, pl.ds(r0, ROWS), :],
                                      tbuf.at[slot], tsem[slot]).start()

            @pl.when(chan == 1)
            def _():
                pltpu.make_async_copy(aff_hbm.at[samp, pl.ds(r0, ROWS), :],
                                      tbuf.at[slot], tsem[slot]).start()

        def wait_load(slot):
            pltpu.make_async_copy(pred_hbm.at[0, 0, pl.ds(0, ROWS), :],
                                  pbuf.at[slot], psem[slot]).wait()
            pltpu.make_async_copy(char_hbm.at[0, pl.ds(0, ROWS), :],
                                  tbuf.at[slot], tsem[slot]).wait()

        def compute(slot, carry):
            def row_step(r, c2):
                aa, ap, ct = c2
                ls, ms = [], []
                for u in range(W // L):
                    p = pbuf[slot, r, pl.ds(u * L, L)]
                    t = tbuf[slot, r, pl.ds(u * L, L)]
                    d = p - t
                    ls.append(d * d)
                    ms.append(t > 0.0)
                lp = [jnp.where(m, l, 0.0) for m, l in zip(ms, ls)]
                lc = [jnp.where(m, 1.0, 0.0) for m in ms]
                def tree(xs):
                    while len(xs) > 1:
                        xs = [xs[j] + xs[j + 1] for j in range(0, len(xs), 2)]
                    return xs[0]
                return (aa + tree(ls), ap + tree(lp), ct + tree(lc))

            return lax.fori_loop(0, ROWS, row_step, carry)

        zeros = jnp.zeros((L,), jnp.float32)
        carry = (zeros, zeros, zeros)
        start_load(0, 0)
        def super_body(gg, carry):
            start_load(2 * gg + 1, 1)
            wait_load(0)
            carry = compute(0, carry)

            @pl.when(gg < NCHUNK // 2 - 1)
            def _():
                start_load(2 * gg + 2, 0)

            wait_load(1)
            return compute(1, carry)

        acc_all, acc_pos, cnt = lax.fori_loop(0, NCHUNK // 2, super_body, carry)
        ostage[0, pl.ds(0, L)] = acc_all
        ostage[1, pl.ds(0, L)] = acc_pos
        ostage[2, pl.ds(0, L)] = cnt
        pltpu.sync_copy(ostage, out_hbm.at[w])

    return reduce_kernel


_reduce = _make_reduce_kernel()


def _topk_tc_kernel(k_sref, pred_ref, tgt_ref, out_ref):
    w = pl.program_id(0)
    k = k_sref[w]
    pred = pred_ref[0]
    tgt = tgt_ref[0]
    loss = (pred - tgt) ** 2
    keys = jnp.where(tgt > 0.0, jnp.int32(-1),
                     lax.bitcast_convert_type(loss, jnp.int32))

    def bit_step(i, t):
        cand = t | (jnp.int32(1) << (30 - i))
        cnt = jnp.sum((keys >= cand).astype(jnp.int32))
        return jnp.where(cnt >= k, cand, t)

    t = lax.fori_loop(0, 31, bit_step, jnp.int32(0))
    cnt_gt = jnp.sum((keys > t).astype(jnp.int32))
    sum_gt = jnp.sum(jnp.where(keys > t, loss, 0.0))
    tval = lax.bitcast_convert_type(t, jnp.float32)
    res = sum_gt + (k - cnt_gt).astype(jnp.float32) * tval
    out_ref[0] = jnp.full((8, 128), res, jnp.float32)


def _exact_topk_sums(kk, pred32, char_target, aff_target):
    tgt32 = jnp.stack([char_target, aff_target], axis=1).reshape(NW, H, W)
    grid_spec = pltpu.PrefetchScalarGridSpec(
        num_scalar_prefetch=1,
        grid=(NW,),
        in_specs=[
            pl.BlockSpec((1, H, W), lambda w, k: (w, 0, 0)),
            pl.BlockSpec((1, H, W), lambda w, k: (w, 0, 0)),
        ],
        out_specs=pl.BlockSpec((1, 8, 128), lambda w, k: (w, 0, 0)),
    )
    out = pl.pallas_call(
        _topk_tc_kernel,
        grid_spec=grid_spec,
        out_shape=jax.ShapeDtypeStruct((NW, 8, 128), jnp.float32),
    )(kk, pred32.reshape(NW, H, W), tgt32)
    return out[:, 0, 0]


def kernel(output_imgs, char_target, aff_target):
    B = output_imgs.shape[0]
    pred32 = output_imgs.reshape(NW, N)

    accs = _reduce(output_imgs, char_target, aff_target)
    sums = jnp.sum(accs[:, :3, :L], axis=-1)
    sum_all = sums[:, 0]
    sum_pos = sums[:, 1]
    num_pos_f = sums[:, 2]
    num_pos = num_pos_f.astype(jnp.int32)

    m = jnp.int32(N) - num_pos
    k = jnp.minimum(num_pos * 3, m)
    kf = k.astype(jnp.float32)

    mean_all = sum_all / jnp.float32(N)
    positive_mean = sum_pos / num_pos_f
    easy_topk_mean = (sum_all - sum_pos) / kf

    need_hard = jnp.any((num_pos * 3 < m) & (k >= 10))
    hard_sums = lax.cond(
        need_hard,
        lambda: _exact_topk_sums(k, pred32, char_target, aff_target),
        lambda: jnp.zeros((NW,), jnp.float32),
    )
    topk_mean = jnp.where(num_pos * 3 >= m, easy_topk_mean, hard_sums / kf)
    ohem = positive_mean + topk_mean
    pair_loss = jnp.where(k < 10, mean_all, ohem)
    return jnp.sum(pair_loss) / jnp.float32(B)


# speedup vs baseline: 86.5551x; 1.1286x over previous
import functools

import jax
import jax.numpy as jnp
from jax import lax
from jax.experimental import pallas as pl
from jax.experimental.pallas import tpu as pltpu
from jax.experimental.pallas import tpu_sc as plsc

NC, NS, L = 2, 16, 16
NW = NC * NS
H = W = 512
N = H * W
ROWS = 32
NCHUNK = H // ROWS


def _make_reduce_kernel():
    mesh = plsc.VectorSubcoreMesh(
        core_axis_name="c", subcore_axis_name="s",
        num_cores=NC, num_subcores=NS)

    @functools.partial(
        pl.kernel,
        out_type=jax.ShapeDtypeStruct((NW, 8, 128), jnp.float32),
        mesh=mesh,
        compiler_params=pltpu.CompilerParams(use_tc_tiling_on_sc=True,
                                             needs_layout_passes=False),
        scratch_types=[
            pltpu.VMEM((2, ROWS, W), jnp.float32),
            pltpu.VMEM((2, ROWS, W), jnp.float32),
            pltpu.VMEM((8, 128), jnp.float32),
            pltpu.SemaphoreType.DMA,
            pltpu.SemaphoreType.DMA,
            pltpu.SemaphoreType.DMA,
            pltpu.SemaphoreType.DMA,
        ],
    )
    def reduce_kernel(pred_hbm, char_hbm, aff_hbm, out_hbm, pbuf, tbuf,
                      ostage, ps0, ps1, ts0, ts1):
        w = lax.axis_index("s") * NC + lax.axis_index("c")
        samp = w // 2
        chan = w % 2
        psem = (ps0, ps1)
        tsem = (ts0, ts1)

        def start_load(g, slot):
            r0 = g * ROWS
            pltpu.make_async_copy(pred_hbm.at[samp, chan, pl.ds(r0, ROWS), :],
                                  pbuf.at[slot], psem[slot]).start()

            @pl.when(chan == 0)
            def _():
                pltpu.make_async_copy(char_hbm.at[samp, pl.ds(r0, ROWS), :],
                                      tbuf.at[slot], tsem[slot]).start()

            @pl.when(chan == 1)
            def _():
                pltpu.make_async_copy(aff_hbm.at[samp, pl.ds(r0, ROWS), :],
                                      tbuf.at[slot], tsem[slot]).start()

        def wait_load(slot):
            pltpu.make_async_copy(pred_hbm.at[0, 0, pl.ds(0, ROWS), :],
                                  pbuf.at[slot], psem[slot]).wait()
            pltpu.make_async_copy(char_hbm.at[0, pl.ds(0, ROWS), :],
                                  tbuf.at[slot], tsem[slot]).wait()

        GROUP = 8

        def tree(xs):
            while len(xs) > 1:
                xs = [xs[j] + xs[j + 1] for j in range(0, len(xs), 2)]
            return xs[0]

        def compute(slot, carry):
            def row_step(r, c2):
                aa, ap, ct = c2
                for g in range(W // L // GROUP):
                    ls, ms = [], []
                    for u in range(GROUP):
                        off = (g * GROUP + u) * L
                        p = pbuf[slot, r, pl.ds(off, L)]
                        t = tbuf[slot, r, pl.ds(off, L)]
                        d = p - t
                        ls.append(d * d)
                        ms.append(t > 0.0)
                    aa = aa + tree(ls)
                    ap = ap + tree([jnp.where(m, l, 0.0)
                                    for m, l in zip(ms, ls)])
                    ct = ct + tree([plsc.all_reduce_population_count(m)
                                    for m in ms])
                return (aa, ap, ct)

            return lax.fori_loop(0, ROWS, row_step, carry)

        zeros = jnp.zeros((L,), jnp.float32)
        carry = (zeros, zeros, jnp.zeros((L,), jnp.int32))
        start_load(0, 0)
        def super_body(gg, carry):
            start_load(2 * gg + 1, 1)
            wait_load(0)
            carry = compute(0, carry)

            @pl.when(gg < NCHUNK // 2 - 1)
            def _():
                start_load(2 * gg + 2, 0)

            wait_load(1)
            return compute(1, carry)

        acc_all, acc_pos, cnt = lax.fori_loop(0, NCHUNK // 2, super_body, carry)
        ostage[0, pl.ds(0, L)] = acc_all
        ostage[1, pl.ds(0, L)] = acc_pos
        ostage[2, pl.ds(0, L)] = cnt.astype(jnp.float32)
        pltpu.sync_copy(ostage, out_hbm.at[w])

    return reduce_kernel


_reduce = _make_reduce_kernel()


def _topk_tc_kernel(k_sref, pred_ref, tgt_ref, out_ref):
    w = pl.program_id(0)
    k = k_sref[w]
    pred = pred_ref[0]
    tgt = tgt_ref[0]
    loss = (pred - tgt) ** 2
    keys = jnp.where(tgt > 0.0, jnp.int32(-1),
                     lax.bitcast_convert_type(loss, jnp.int32))

    def bit_step(i, t):
        cand = t | (jnp.int32(1) << (30 - i))
        cnt = jnp.sum((keys >= cand).astype(jnp.int32))
        return jnp.where(cnt >= k, cand, t)

    t = lax.fori_loop(0, 31, bit_step, jnp.int32(0))
    cnt_gt = jnp.sum((keys > t).astype(jnp.int32))
    sum_gt = jnp.sum(jnp.where(keys > t, loss, 0.0))
    tval = lax.bitcast_convert_type(t, jnp.float32)
    res = sum_gt + (k - cnt_gt).astype(jnp.float32) * tval
    out_ref[0] = jnp.full((8, 128), res, jnp.float32)


def _exact_topk_sums(kk, pred32, char_target, aff_target):
    tgt32 = jnp.stack([char_target, aff_target], axis=1).reshape(NW, H, W)
    grid_spec = pltpu.PrefetchScalarGridSpec(
        num_scalar_prefetch=1,
        grid=(NW,),
        in_specs=[
            pl.BlockSpec((1, H, W), lambda w, k: (w, 0, 0)),
            pl.BlockSpec((1, H, W), lambda w, k: (w, 0, 0)),
        ],
        out_specs=pl.BlockSpec((1, 8, 128), lambda w, k: (w, 0, 0)),
    )
    out = pl.pallas_call(
        _topk_tc_kernel,
        grid_spec=grid_spec,
        out_shape=jax.ShapeDtypeStruct((NW, 8, 128), jnp.float32),
    )(kk, pred32.reshape(NW, H, W), tgt32)
    return out[:, 0, 0]


def kernel(output_imgs, char_target, aff_target):
    B = output_imgs.shape[0]
    pred32 = output_imgs.reshape(NW, N)

    accs = _reduce(output_imgs, char_target, aff_target)
    sums = jnp.sum(accs[:, :3, :L], axis=-1)
    sum_all = sums[:, 0]
    sum_pos = sums[:, 1]
    num_pos_f = sums[:, 2] / jnp.float32(L)
    num_pos = num_pos_f.astype(jnp.int32)

    m = jnp.int32(N) - num_pos
    k = jnp.minimum(num_pos * 3, m)
    kf = k.astype(jnp.float32)

    mean_all = sum_all / jnp.float32(N)
    positive_mean = sum_pos / num_pos_f
    easy_topk_mean = (sum_all - sum_pos) / kf

    need_hard = jnp.any((num_pos * 3 < m) & (k >= 10))
    hard_sums = lax.cond(
        need_hard,
        lambda: _exact_topk_sums(k, pred32, char_target, aff_target),
        lambda: jnp.zeros((NW,), jnp.float32),
    )
    topk_mean = jnp.where(num_pos * 3 >= m, easy_topk_mean, hard_sums / kf)
    ohem = positive_mean + topk_mean
    pair_loss = jnp.where(k < 10, mean_all, ohem)
    return jnp.sum(pair_loss) / jnp.float32(B)


# speedup vs baseline: 133.6329x; 1.5439x over previous
import functools

import jax
import jax.numpy as jnp
from jax import lax
from jax.experimental import pallas as pl
from jax.experimental.pallas import tpu as pltpu
from jax.experimental.pallas import tpu_sc as plsc

NC, NS, L = 2, 16, 16
NW = NC * NS
H = W = 512
N = H * W
ROWS = 32
NCHUNK = H // ROWS


def _make_reduce_kernel():
    mesh = plsc.VectorSubcoreMesh(
        core_axis_name="c", subcore_axis_name="s",
        num_cores=NC, num_subcores=NS)

    @functools.partial(
        pl.kernel,
        out_type=jax.ShapeDtypeStruct((NW, 8, 128), jnp.float32),
        mesh=mesh,
        compiler_params=pltpu.CompilerParams(use_tc_tiling_on_sc=True,
                                             needs_layout_passes=False),
        scratch_types=[
            pltpu.VMEM((2, ROWS, W), jnp.float32),
            pltpu.VMEM((2, ROWS, W), jnp.float32),
            pltpu.VMEM((8, 128), jnp.float32),
            pltpu.SemaphoreType.DMA,
            pltpu.SemaphoreType.DMA,
            pltpu.SemaphoreType.DMA,
            pltpu.SemaphoreType.DMA,
        ],
    )
    def reduce_kernel(pred_hbm, char_hbm, aff_hbm, out_hbm, pbuf, tbuf,
                      ostage, ps0, ps1, ts0, ts1):
        w = lax.axis_index("s") * NC + lax.axis_index("c")
        samp = w // 2
        chan = w % 2
        psem = (ps0, ps1)
        tsem = (ts0, ts1)

        def start_load(g, slot):
            r0 = g * ROWS
            pltpu.make_async_copy(pred_hbm.at[samp, chan, pl.ds(r0, ROWS), :],
                                  pbuf.at[slot], psem[slot]).start()

            @pl.when(chan == 0)
            def _():
                pltpu.make_async_copy(char_hbm.at[samp, pl.ds(r0, ROWS), :],
                                      tbuf.at[slot], tsem[slot]).start()

            @pl.when(chan == 1)
            def _():
                pltpu.make_async_copy(aff_hbm.at[samp, pl.ds(r0, ROWS), :],
                                      tbuf.at[slot], tsem[slot]).start()

        def wait_load(slot):
            pltpu.make_async_copy(pred_hbm.at[0, 0, pl.ds(0, ROWS), :],
                                  pbuf.at[slot], psem[slot]).wait()
            pltpu.make_async_copy(char_hbm.at[0, pl.ds(0, ROWS), :],
                                  tbuf.at[slot], tsem[slot]).wait()

        GROUP = 8

        def tree(xs):
            while len(xs) > 1:
                xs = [xs[j] + xs[j + 1] for j in range(0, len(xs), 2)]
            return xs[0]

        GROUPS_PER_ROW = W // L // GROUP

        def compute(slot, carry):
            def gstep(i, c2):
                aa, ap, ct = c2
                r = i >> 2
                base = (i & 3) * (GROUP * L)
                ls, ms = [], []
                for u in range(GROUP):
                    off = base + u * L
                    p = pbuf[slot, r, pl.ds(off, L)]
                    t = tbuf[slot, r, pl.ds(off, L)]
                    d = p - t
                    ls.append(d * d)
                    ms.append(t > 0.0)
                aa = aa + tree(ls)
                ap = ap + tree([jnp.where(m, l, 0.0)
                                for m, l in zip(ms, ls)])
                ct = ct + tree([plsc.all_reduce_population_count(m)
                                for m in ms])
                return (aa, ap, ct)

            return lax.fori_loop(0, ROWS * GROUPS_PER_ROW, gstep, carry)

        zeros = jnp.zeros((L,), jnp.float32)
        carry = (zeros, zeros, jnp.zeros((L,), jnp.int32))
        start_load(0, 0)
        def super_body(gg, carry):
            start_load(2 * gg + 1, 1)
            wait_load(0)
            carry = compute(0, carry)

            @pl.when(gg < NCHUNK // 2 - 1)
            def _():
                start_load(2 * gg + 2, 0)

            wait_load(1)
            return compute(1, carry)

        acc_all, acc_pos, cnt = lax.fori_loop(0, NCHUNK // 2, super_body, carry)
        ostage[0, pl.ds(0, L)] = acc_all
        ostage[1, pl.ds(0, L)] = acc_pos
        ostage[2, pl.ds(0, L)] = cnt.astype(jnp.float32)
        pltpu.sync_copy(ostage, out_hbm.at[w])

    return reduce_kernel


_reduce = _make_reduce_kernel()


def _topk_tc_kernel(k_sref, pred_ref, tgt_ref, out_ref):
    w = pl.program_id(0)
    k = k_sref[w]
    pred = pred_ref[0]
    tgt = tgt_ref[0]
    loss = (pred - tgt) ** 2
    keys = jnp.where(tgt > 0.0, jnp.int32(-1),
                     lax.bitcast_convert_type(loss, jnp.int32))

    def bit_step(i, t):
        cand = t | (jnp.int32(1) << (30 - i))
        cnt = jnp.sum((keys >= cand).astype(jnp.int32))
        return jnp.where(cnt >= k, cand, t)

    t = lax.fori_loop(0, 31, bit_step, jnp.int32(0))
    cnt_gt = jnp.sum((keys > t).astype(jnp.int32))
    sum_gt = jnp.sum(jnp.where(keys > t, loss, 0.0))
    tval = lax.bitcast_convert_type(t, jnp.float32)
    res = sum_gt + (k - cnt_gt).astype(jnp.float32) * tval
    out_ref[0] = jnp.full((8, 128), res, jnp.float32)


def _exact_topk_sums(kk, pred32, char_target, aff_target):
    tgt32 = jnp.stack([char_target, aff_target], axis=1).reshape(NW, H, W)
    grid_spec = pltpu.PrefetchScalarGridSpec(
        num_scalar_prefetch=1,
        grid=(NW,),
        in_specs=[
            pl.BlockSpec((1, H, W), lambda w, k: (w, 0, 0)),
            pl.BlockSpec((1, H, W), lambda w, k: (w, 0, 0)),
        ],
        out_specs=pl.BlockSpec((1, 8, 128), lambda w, k: (w, 0, 0)),
    )
    out = pl.pallas_call(
        _topk_tc_kernel,
        grid_spec=grid_spec,
        out_shape=jax.ShapeDtypeStruct((NW, 8, 128), jnp.float32),
    )(kk, pred32.reshape(NW, H, W), tgt32)
    return out[:, 0, 0]


def kernel(output_imgs, char_target, aff_target):
    B = output_imgs.shape[0]
    pred32 = output_imgs.reshape(NW, N)

    accs = _reduce(output_imgs, char_target, aff_target)
    sums = jnp.sum(accs[:, :3, :L], axis=-1)
    sum_all = sums[:, 0]
    sum_pos = sums[:, 1]
    num_pos_f = sums[:, 2] / jnp.float32(L)
    num_pos = num_pos_f.astype(jnp.int32)

    m = jnp.int32(N) - num_pos
    k = jnp.minimum(num_pos * 3, m)
    kf = k.astype(jnp.float32)

    mean_all = sum_all / jnp.float32(N)
    positive_mean = sum_pos / num_pos_f
    easy_topk_mean = (sum_all - sum_pos) / kf

    need_hard = jnp.any((num_pos * 3 < m) & (k >= 10))
    hard_sums = lax.cond(
        need_hard,
        lambda: _exact_topk_sums(k, pred32, char_target, aff_target),
        lambda: jnp.zeros((NW,), jnp.float32),
    )
    topk_mean = jnp.where(num_pos * 3 >= m, easy_topk_mean, hard_sums / kf)
    ohem = positive_mean + topk_mean
    pair_loss = jnp.where(k < 10, mean_all, ohem)
    return jnp.sum(pair_loss) / jnp.float32(B)
